# baseline (speedup 1.0000x reference)
import functools

import jax
import jax.numpy as jnp
from jax import lax
from jax.experimental import pallas as pl
from jax.experimental.pallas import tpu as pltpu
from jax.experimental.pallas import tpu_sc as plsc

N = 10000
E = 320000
D = 128
NC = 2
NS = 16
NW = NC * NS
K = 128
SLOW_C = 1
CSLOW = 80
CFAST = 80
G = NS * (CSLOW + CFAST)
EP = G * K
CHD = G // NW
NP = 10240
PT = NP // NS
RB = 1024
GRID = (N + RB - 1) // RB

_mesh = plsc.VectorSubcoreMesh(core_axis_name="c", subcore_axis_name="s")


@functools.partial(
    pl.kernel,
    out_type=jax.ShapeDtypeStruct((NC * NP,), jnp.float32),
    mesh=_mesh,
    scratch_types=[
        pltpu.VMEM((CHD, K), jnp.int32),
        pltpu.VMEM((K,), jnp.float32),
        pltpu.VMEM((PT,), jnp.float32),
        pltpu.VMEM_SHARED((NP,), jnp.float32),
    ],
)
def _deg_kernel(dst_hbm, zdeg_hbm, ones_hbm, out_hbm, dst_v, ones_v, stage_v, acc_s):
    c = lax.axis_index("c")
    s = lax.axis_index("s")
    wid = c * NS + s
    pltpu.sync_copy(dst_hbm.at[pl.ds(pl.multiple_of(wid * CHD, 8), CHD)], dst_v)
    pltpu.sync_copy(ones_hbm, ones_v)
    pltpu.sync_copy(zdeg_hbm, stage_v)
    pltpu.sync_copy(stage_v, acc_s.at[pl.ds(s * PT, PT)])
    plsc.subcore_barrier()

    def body(j, carry):
        pltpu.sync_copy(ones_v, acc_s.at[dst_v.at[j]], add=True)
        return carry

    lax.fori_loop(0, CHD, body, 0)
    plsc.subcore_barrier()
    pltpu.sync_copy(acc_s.at[pl.ds(s * PT, PT)], stage_v)
    pltpu.sync_copy(stage_v, out_hbm.at[pl.ds(c * NP + s * PT, PT)])


@functools.partial(
    pl.kernel,
    out_type=jax.ShapeDtypeStruct((NC, NP, D), jnp.float32),
    mesh=_mesh,
    scratch_types=[
        pltpu.VMEM((2, K), jnp.int32),
        pltpu.VMEM((CFAST, K), jnp.int32),
        pltpu.VMEM((K, D), jnp.float32),
        pltpu.VMEM((K, D), jnp.float32),
        pltpu.VMEM_SHARED((NP, D), jnp.float32),
        pltpu.SemaphoreType.DMA,
        pltpu.SemaphoreType.DMA,
        pltpu.SemaphoreType.DMA,
        pltpu.SemaphoreType.DMA,
    ],
)
def _edge_kernel(y_hbm, src_hbm, dst_hbm, zrows_hbm, out_hbm,
                 srcb_v, dst_v, rows0_v, rows1_v, acc_s,
                 sem0, sem1, semS0, semS1):
    c = lax.axis_index("c")
    s = lax.axis_index("s")
    rows = (rows0_v, rows1_v)
    sems = (sem0, sem1)
    semS = (semS0, semS1)
    pltpu.sync_copy(zrows_hbm, rows0_v)
    for k in range(PT // K):
        pltpu.sync_copy(rows0_v, acc_s.at[pl.ds(s * PT + k * K, K)])

    def wait_gather(b):
        pltpu.make_async_copy(y_hbm.at[srcb_v.at[b]], rows[b], sems[b]).wait()

    def wait_srcrow(b):
        pltpu.make_async_copy(src_hbm.at[0], srcb_v.at[b], semS[b]).wait()

    def pipeline(base, chl):
        pltpu.sync_copy(dst_hbm.at[pl.ds(base, chl)], dst_v.at[pl.ds(0, chl)])
        pltpu.sync_copy(src_hbm.at[base], srcb_v.at[0])
        pltpu.async_copy(y_hbm.at[srcb_v.at[0]], rows0_v, sem0)
        pltpu.async_copy(src_hbm.at[base + 1], srcb_v.at[1], semS1)
        plsc.subcore_barrier()

        def body(h, carry):
            j0 = 2 * h
            wait_gather(0)
            wait_srcrow(1)
            pltpu.async_copy(y_hbm.at[srcb_v.at[1]], rows1_v, sem1)
            pltpu.async_copy(src_hbm.at[base + j0 + 2], srcb_v.at[0], semS0)
            pltpu.sync_copy(rows0_v, acc_s.at[dst_v.at[j0]], add=True)
            wait_gather(1)
            wait_srcrow(0)
            pltpu.async_copy(y_hbm.at[srcb_v.at[0]], rows0_v, sem0)
            pltpu.async_copy(src_hbm.at[base + j0 + 3], srcb_v.at[1], semS1)
            pltpu.sync_copy(rows1_v, acc_s.at[dst_v.at[j0 + 1]], add=True)
            return carry

        lax.fori_loop(0, chl // 2, body, 0)
        wait_gather(0)
        wait_srcrow(1)
        plsc.subcore_barrier()
        for k in range(PT // K):
            b = k % 2
            if k >= 2:
                pltpu.make_async_copy(
                    rows[b], out_hbm.at[c, pl.ds(s * PT + (k - 2) * K, K)],
                    sems[b]).wait()
            pltpu.sync_copy(acc_s.at[pl.ds(s * PT + k * K, K)], rows[b])
            pltpu.async_copy(rows[b], out_hbm.at[c, pl.ds(s * PT + k * K, K)],
                             sems[b])
        for k in (PT // K - 2, PT // K - 1):
            pltpu.make_async_copy(
                rows[k % 2], out_hbm.at[c, pl.ds(s * PT + k * K, K)],
                sems[k % 2]).wait()

    nslow = NS * CSLOW
    if SLOW_C == 0:
        base0, chl0, base1, chl1 = s * CSLOW, CSLOW, nslow + s * CFAST, CFAST
    else:
        base0, chl0, base1, chl1 = s * CFAST, CFAST, NS * CFAST + s * CSLOW, CSLOW

    @pl.when(c == 0)
    def _():
        pipeline(pl.multiple_of(base0, 8), chl0)

    @pl.when(c == 1)
    def _():
        pipeline(pl.multiple_of(base1, 8), chl1)


def _feat_body(x_ref, w_ref, b_ref, o_ref):
    acc = lax.dot_general(x_ref[...], w_ref[...], (((1,), (1,)), ((), ())),
                          preferred_element_type=jnp.float32)
    o_ref[...] = jnp.maximum(acc + b_ref[...], 0.0)


_feat_call = pl.pallas_call(
    _feat_body,
    grid=(GRID,),
    in_specs=[
        pl.BlockSpec((RB, D), lambda i: (i, 0)),
        pl.BlockSpec((D, D), lambda i: (0, 0)),
        pl.BlockSpec((D,), lambda i: (0,)),
    ],
    out_specs=pl.BlockSpec((RB, D), lambda i: (i, 0)),
    out_shape=jax.ShapeDtypeStruct((N, D), jnp.float32),
)


def _dinv_slice(dp_ref):
    i = pl.program_id(0)
    d0 = dp_ref[pl.ds(i * RB, RB)]
    d1 = dp_ref[pl.ds(NP + i * RB, RB)]
    return lax.rsqrt(1.0 + d0 + d1)


def _y1_body(h_ref, w_ref, dp_ref, o_ref):
    dv = _dinv_slice(dp_ref)
    acc = lax.dot_general(h_ref[...], w_ref[...], (((1,), (1,)), ((), ())),
                          preferred_element_type=jnp.float32)
    o_ref[...] = acc * dv[:, None]


_y1_call = pl.pallas_call(
    _y1_body,
    grid=(GRID,),
    in_specs=[
        pl.BlockSpec((RB, D), lambda i: (i, 0)),
        pl.BlockSpec((D, D), lambda i: (0, 0)),
        pl.BlockSpec((NC * NP,), lambda i: (0,)),
    ],
    out_specs=pl.BlockSpec((RB, D), lambda i: (i, 0)),
    out_shape=jax.ShapeDtypeStruct((N, D), jnp.float32),
)


def _mid_body(a_ref, y_ref, dp_ref, b_ref, w_ref, o_ref):
    dv = _dinv_slice(dp_ref)
    h = dv[:, None] * (a_ref[0] + a_ref[1] + y_ref[...]) + b_ref[...]
    acc = lax.dot_general(h, w_ref[...], (((1,), (1,)), ((), ())),
                          preferred_element_type=jnp.float32)
    o_ref[...] = acc * dv[:, None]


_mid_call = pl.pallas_call(
    _mid_body,
    grid=(GRID,),
    in_specs=[
        pl.BlockSpec((NC, RB, D), lambda i: (0, i, 0)),
        pl.BlockSpec((RB, D), lambda i: (i, 0)),
        pl.BlockSpec((NC * NP,), lambda i: (0,)),
        pl.BlockSpec((D,), lambda i: (0,)),
        pl.BlockSpec((D, D), lambda i: (0, 0)),
    ],
    out_specs=pl.BlockSpec((RB, D), lambda i: (i, 0)),
    out_shape=jax.ShapeDtypeStruct((N, D), jnp.float32),
)


def _out_body(a_ref, y_ref, dp_ref, b_ref, o_ref):
    dv = _dinv_slice(dp_ref)
    o_ref[...] = dv[:, None] * (a_ref[0] + a_ref[1] + y_ref[...]) + b_ref[...]


_out_call = pl.pallas_call(
    _out_body,
    grid=(GRID,),
    in_specs=[
        pl.BlockSpec((NC, RB, D), lambda i: (0, i, 0)),
        pl.BlockSpec((RB, D), lambda i: (i, 0)),
        pl.BlockSpec((NC * NP,), lambda i: (0,)),
        pl.BlockSpec((D,), lambda i: (0,)),
    ],
    out_specs=pl.BlockSpec((RB, D), lambda i: (i, 0)),
    out_shape=jax.ShapeDtypeStruct((N, D), jnp.float32),
)


def kernel(x, edge_index, W_in, b_in, W1, b1, W2, b2):
    pad = EP - E
    src = jnp.concatenate([edge_index[0], jnp.zeros((pad,), jnp.int32)])
    pad_dst = N + (jnp.arange(pad, dtype=jnp.int32) % (NP - N))
    dst = jnp.concatenate([edge_index[1], pad_dst])
    src3 = jnp.concatenate(
        [src.reshape(G, K), jnp.zeros((2, K), jnp.int32)], axis=0)
    dst3 = dst.reshape(G, K)
    zdeg = jnp.zeros((PT,), jnp.float32)
    ones = jnp.ones((K,), jnp.float32)
    zrows = jnp.zeros((K, D), jnp.float32)

    deg_p = _deg_kernel(dst3, zdeg, ones)
    feat = _feat_call(x, W_in, b_in)

    y1 = _y1_call(feat, W1, deg_p)
    a1 = _edge_kernel(y1, src3, dst3, zrows)
    y2 = _mid_call(a1, y1, deg_p, b1, W2)
    a2 = _edge_kernel(y2, src3, dst3, zrows)
    out = _out_call(a2, y2, deg_p, b2)
    return (feat, out)


# speedup vs baseline: 1.0034x; 1.0034x over previous
import functools

import jax
import jax.numpy as jnp
from jax import lax
from jax.experimental import pallas as pl
from jax.experimental.pallas import tpu as pltpu
from jax.experimental.pallas import tpu_sc as plsc

N = 10000
E = 320000
D = 128
NC = 2
NS = 16
NW = NC * NS
K = 128
SLOW_C = 1
CSLOW = 80
CFAST = 80
G = NS * (CSLOW + CFAST)
EP = G * K
CHD = G // NW
NP = 10240
PT = NP // NS
RB = 1024
GRID = (N + RB - 1) // RB

_mesh = plsc.VectorSubcoreMesh(core_axis_name="c", subcore_axis_name="s")


@functools.partial(
    pl.kernel,
    out_type=jax.ShapeDtypeStruct((NC * NP,), jnp.float32),
    mesh=_mesh,
    scratch_types=[
        pltpu.VMEM((CHD, K), jnp.int32),
        pltpu.VMEM((K,), jnp.float32),
        pltpu.VMEM((PT,), jnp.float32),
        pltpu.VMEM_SHARED((NP,), jnp.float32),
    ],
)
def _deg_kernel(dst_hbm, zdeg_hbm, ones_hbm, out_hbm, dst_v, ones_v, stage_v, acc_s):
    c = lax.axis_index("c")
    s = lax.axis_index("s")
    wid = c * NS + s
    pltpu.sync_copy(dst_hbm.at[pl.ds(pl.multiple_of(wid * CHD, 8), CHD)], dst_v)
    pltpu.sync_copy(ones_hbm, ones_v)
    pltpu.sync_copy(zdeg_hbm, stage_v)
    pltpu.sync_copy(stage_v, acc_s.at[pl.ds(s * PT, PT)])
    plsc.subcore_barrier()

    def body(j, carry):
        pltpu.sync_copy(ones_v, acc_s.at[dst_v.at[j]], add=True)
        return carry

    lax.fori_loop(0, CHD, body, 0)
    plsc.subcore_barrier()
    pltpu.sync_copy(acc_s.at[pl.ds(s * PT, PT)], stage_v)
    pltpu.sync_copy(stage_v, out_hbm.at[pl.ds(c * NP + s * PT, PT)])


@functools.partial(
    pl.kernel,
    out_type=jax.ShapeDtypeStruct((NC, NP, D), jnp.float32),
    mesh=_mesh,
    scratch_types=[
        pltpu.VMEM((2, K), jnp.int32),
        pltpu.VMEM((CFAST, K), jnp.int32),
        pltpu.VMEM((K, D), jnp.float32),
        pltpu.VMEM((K, D), jnp.float32),
        pltpu.VMEM_SHARED((NP, D), jnp.float32),
        pltpu.SemaphoreType.DMA,
        pltpu.SemaphoreType.DMA,
        pltpu.SemaphoreType.DMA,
        pltpu.SemaphoreType.DMA,
    ],
)
def _edge_kernel(y_hbm, src_hbm, dst_hbm, zrows_hbm, out_hbm,
                 srcb_v, dst_v, rows0_v, rows1_v, acc_s,
                 sem0, sem1, semS0, semS1):
    c = lax.axis_index("c")
    s = lax.axis_index("s")
    rows = (rows0_v, rows1_v)
    sems = (sem0, sem1)
    semS = (semS0, semS1)
    pltpu.sync_copy(zrows_hbm, rows0_v)
    for k in range(PT // K):
        pltpu.sync_copy(rows0_v, acc_s.at[pl.ds(s * PT + k * K, K)])

    def wait_gather(b):
        pltpu.make_async_copy(y_hbm.at[srcb_v.at[b]], rows[b], sems[b]).wait()

    def wait_srcrow(b):
        pltpu.make_async_copy(src_hbm.at[0], srcb_v.at[b], semS[b]).wait()

    def pipeline(base, chl):
        pltpu.sync_copy(dst_hbm.at[pl.ds(base, chl)], dst_v.at[pl.ds(0, chl)])
        pltpu.sync_copy(src_hbm.at[base], srcb_v.at[0])
        pltpu.async_copy(y_hbm.at[srcb_v.at[0]], rows0_v, sem0)
        pltpu.async_copy(src_hbm.at[base + 1], srcb_v.at[1], semS1)
        plsc.subcore_barrier()

        def body(h, carry):
            j0 = 2 * h
            wait_gather(0)
            wait_srcrow(1)
            pltpu.async_copy(y_hbm.at[srcb_v.at[1]], rows1_v, sem1)
            pltpu.async_copy(src_hbm.at[base + j0 + 2], srcb_v.at[0], semS0)
            pltpu.sync_copy(rows0_v, acc_s.at[dst_v.at[j0]], add=True)
            wait_gather(1)
            wait_srcrow(0)
            pltpu.async_copy(y_hbm.at[srcb_v.at[0]], rows0_v, sem0)
            pltpu.async_copy(src_hbm.at[base + j0 + 3], srcb_v.at[1], semS1)
            pltpu.sync_copy(rows1_v, acc_s.at[dst_v.at[j0 + 1]], add=True)
            return carry

        lax.fori_loop(0, chl // 2, body, 0)
        wait_gather(0)
        wait_srcrow(1)
        plsc.subcore_barrier()
        for k in range(PT // K):
            b = k % 2
            if k >= 2:
                pltpu.make_async_copy(
                    rows[b], out_hbm.at[c, pl.ds(s * PT + (k - 2) * K, K)],
                    sems[b]).wait()
            pltpu.sync_copy(acc_s.at[pl.ds(s * PT + k * K, K)], rows[b])
            pltpu.async_copy(rows[b], out_hbm.at[c, pl.ds(s * PT + k * K, K)],
                             sems[b])
        for k in (PT // K - 2, PT // K - 1):
            pltpu.make_async_copy(
                rows[k % 2], out_hbm.at[c, pl.ds(s * PT + k * K, K)],
                sems[k % 2]).wait()

    wid = c * NS + s
    pipeline(pl.multiple_of(wid * CSLOW, 8), CSLOW)


def _feat_body(x_ref, w_ref, b_ref, o_ref):
    acc = lax.dot_general(x_ref[...], w_ref[...], (((1,), (1,)), ((), ())),
                          preferred_element_type=jnp.float32)
    o_ref[...] = jnp.maximum(acc + b_ref[...], 0.0)


_feat_call = pl.pallas_call(
    _feat_body,
    grid=(GRID,),
    in_specs=[
        pl.BlockSpec((RB, D), lambda i: (i, 0)),
        pl.BlockSpec((D, D), lambda i: (0, 0)),
        pl.BlockSpec((D,), lambda i: (0,)),
    ],
    out_specs=pl.BlockSpec((RB, D), lambda i: (i, 0)),
    out_shape=jax.ShapeDtypeStruct((N, D), jnp.float32),
)


def _dinv_slice(dp_ref):
    i = pl.program_id(0)
    d0 = dp_ref[pl.ds(i * RB, RB)]
    d1 = dp_ref[pl.ds(NP + i * RB, RB)]
    return lax.rsqrt(1.0 + d0 + d1)


def _y1_body(h_ref, w_ref, dp_ref, o_ref):
    dv = _dinv_slice(dp_ref)
    acc = lax.dot_general(h_ref[...], w_ref[...], (((1,), (1,)), ((), ())),
                          preferred_element_type=jnp.float32)
    o_ref[...] = acc * dv[:, None]


_y1_call = pl.pallas_call(
    _y1_body,
    grid=(GRID,),
    in_specs=[
        pl.BlockSpec((RB, D), lambda i: (i, 0)),
        pl.BlockSpec((D, D), lambda i: (0, 0)),
        pl.BlockSpec((NC * NP,), lambda i: (0,)),
    ],
    out_specs=pl.BlockSpec((RB, D), lambda i: (i, 0)),
    out_shape=jax.ShapeDtypeStruct((N, D), jnp.float32),
)


def _mid_body(a_ref, y_ref, dp_ref, b_ref, w_ref, o_ref):
    dv = _dinv_slice(dp_ref)
    h = dv[:, None] * (a_ref[0] + a_ref[1] + y_ref[...]) + b_ref[...]
    acc = lax.dot_general(h, w_ref[...], (((1,), (1,)), ((), ())),
                          preferred_element_type=jnp.float32)
    o_ref[...] = acc * dv[:, None]


_mid_call = pl.pallas_call(
    _mid_body,
    grid=(GRID,),
    in_specs=[
        pl.BlockSpec((NC, RB, D), lambda i: (0, i, 0)),
        pl.BlockSpec((RB, D), lambda i: (i, 0)),
        pl.BlockSpec((NC * NP,), lambda i: (0,)),
        pl.BlockSpec((D,), lambda i: (0,)),
        pl.BlockSpec((D, D), lambda i: (0, 0)),
    ],
    out_specs=pl.BlockSpec((RB, D), lambda i: (i, 0)),
    out_shape=jax.ShapeDtypeStruct((N, D), jnp.float32),
)


def _out_body(a_ref, y_ref, dp_ref, b_ref, o_ref):
    dv = _dinv_slice(dp_ref)
    o_ref[...] = dv[:, None] * (a_ref[0] + a_ref[1] + y_ref[...]) + b_ref[...]


_out_call = pl.pallas_call(
    _out_body,
    grid=(GRID,),
    in_specs=[
        pl.BlockSpec((NC, RB, D), lambda i: (0, i, 0)),
        pl.BlockSpec((RB, D), lambda i: (i, 0)),
        pl.BlockSpec((NC * NP,), lambda i: (0,)),
        pl.BlockSpec((D,), lambda i: (0,)),
    ],
    out_specs=pl.BlockSpec((RB, D), lambda i: (i, 0)),
    out_shape=jax.ShapeDtypeStruct((N, D), jnp.float32),
)


def kernel(x, edge_index, W_in, b_in, W1, b1, W2, b2):
    pad = EP - E
    src = jnp.concatenate([edge_index[0], jnp.zeros((pad,), jnp.int32)])
    pad_dst = N + (jnp.arange(pad, dtype=jnp.int32) % (NP - N))
    dst = jnp.concatenate([edge_index[1], pad_dst])
    src3 = jnp.concatenate(
        [src.reshape(G, K), jnp.zeros((2, K), jnp.int32)], axis=0)
    dst3 = dst.reshape(G, K)
    zdeg = jnp.zeros((PT,), jnp.float32)
    ones = jnp.ones((K,), jnp.float32)
    zrows = jnp.zeros((K, D), jnp.float32)

    deg_p = _deg_kernel(dst3, zdeg, ones)
    feat = _feat_call(x, W_in, b_in)

    y1 = _y1_call(feat, W1, deg_p)
    a1 = _edge_kernel(y1, src3, dst3, zrows)
    y2 = _mid_call(a1, y1, deg_p, b1, W2)
    a2 = _edge_kernel(y2, src3, dst3, zrows)
    out = _out_call(a2, y2, deg_p, b2)
    return (feat, out)


# speedup vs baseline: 1.5445x; 1.5393x over previous
import functools

import jax
import jax.numpy as jnp
from jax import lax
from jax.experimental import pallas as pl
from jax.experimental.pallas import tpu as pltpu
from jax.experimental.pallas import tpu_sc as plsc

N = 10000
E = 320000
D = 128
NC = 2
NS = 16
NW = NC * NS
K = 128
CH = (E + NW * K - 1) // (NW * K)
EP = NW * K * CH
NP = 10240
PT = NP // NS
RB = 1024
GRID = (N + RB - 1) // RB

_mesh = plsc.VectorSubcoreMesh(core_axis_name="c", subcore_axis_name="s")


@functools.partial(
    pl.kernel,
    out_type=jax.ShapeDtypeStruct((NC * NP,), jnp.float32),
    mesh=_mesh,
    scratch_types=[
        pltpu.VMEM((CH, K), jnp.int32),
        pltpu.VMEM((K,), jnp.float32),
        pltpu.VMEM((PT,), jnp.float32),
        pltpu.VMEM_SHARED((NP,), jnp.float32),
    ],
)
def _deg_kernel(dst_hbm, zdeg_hbm, ones_hbm, out_hbm, dst_v, ones_v, stage_v, acc_s):
    c = lax.axis_index("c")
    s = lax.axis_index("s")
    wid = c * NS + s
    pltpu.sync_copy(dst_hbm.at[wid], dst_v)
    pltpu.sync_copy(ones_hbm, ones_v)
    pltpu.sync_copy(zdeg_hbm, stage_v)
    pltpu.sync_copy(stage_v, acc_s.at[pl.ds(s * PT, PT)])
    plsc.subcore_barrier()

    def body(j, carry):
        pltpu.sync_copy(ones_v, acc_s.at[dst_v.at[j]], add=True)
        return carry

    lax.fori_loop(0, CH, body, 0)
    plsc.subcore_barrier()
    pltpu.sync_copy(acc_s.at[pl.ds(s * PT, PT)], stage_v)
    pltpu.sync_copy(stage_v, out_hbm.at[pl.ds(c * NP + s * PT, PT)])


@functools.partial(
    pl.kernel,
    out_type=jax.ShapeDtypeStruct((NC, NP, D), jnp.float32),
    mesh=_mesh,
    scratch_types=[
        pltpu.VMEM((2, K), jnp.int32),
        pltpu.VMEM((CH, K), jnp.int32),
        pltpu.VMEM((K, D), jnp.float32),
        pltpu.VMEM((K, D), jnp.float32),
        pltpu.VMEM_SHARED((NP, D), jnp.float32),
        pltpu.SemaphoreType.DMA,
        pltpu.SemaphoreType.DMA,
        pltpu.SemaphoreType.DMA,
        pltpu.SemaphoreType.DMA,
    ],
)
def _edge_kernel(y_hbm, src_hbm, dst_hbm, zrows_hbm, out_hbm,
                 srcb_v, dst_v, rows0_v, rows1_v, acc_s,
                 sem0, sem1, semS0, semS1):
    c = lax.axis_index("c")
    s = lax.axis_index("s")
    wid = c * NS + s
    rows = (rows0_v, rows1_v)
    sems = (sem0, sem1)
    semS = (semS0, semS1)
    pltpu.sync_copy(dst_hbm.at[wid], dst_v)
    pltpu.sync_copy(zrows_hbm, rows0_v)
    for k in range(PT // K):
        pltpu.sync_copy(rows0_v, acc_s.at[pl.ds(s * PT + k * K, K)])
    pltpu.sync_copy(src_hbm.at[wid, 0], srcb_v.at[0])
    pltpu.async_copy(y_hbm.at[srcb_v.at[0]], rows0_v, sem0)
    pltpu.async_copy(src_hbm.at[wid, 1], srcb_v.at[1], semS1)
    plsc.subcore_barrier()

    def wait_gather(b):
        pltpu.make_async_copy(y_hbm.at[srcb_v.at[b]], rows[b], sems[b]).wait()

    def wait_srcrow(b):
        pltpu.make_async_copy(src_hbm.at[wid, 0], srcb_v.at[b], semS[b]).wait()

    def body(h, carry):
        j0 = 2 * h
        wait_gather(0)
        wait_srcrow(1)
        pltpu.async_copy(y_hbm.at[srcb_v.at[1]], rows1_v, sem1)
        pltpu.async_copy(src_hbm.at[wid, j0 + 2], srcb_v.at[0], semS0)
        pltpu.sync_copy(rows0_v, acc_s.at[dst_v.at[j0]], add=True)
        wait_gather(1)
        wait_srcrow(0)
        pltpu.async_copy(y_hbm.at[srcb_v.at[0]], rows0_v, sem0)
        pltpu.async_copy(src_hbm.at[wid, j0 + 3], srcb_v.at[1], semS1)
        pltpu.sync_copy(rows1_v, acc_s.at[dst_v.at[j0 + 1]], add=True)
        return carry

    lax.fori_loop(0, (CH - 1) // 2, body, 0)
    wait_gather(0)
    pltpu.sync_copy(rows0_v, acc_s.at[dst_v.at[CH - 1]], add=True)
    wait_srcrow(1)
    plsc.subcore_barrier()
    for k in range(PT // K):
        b = k % 2
        if k >= 2:
            pltpu.make_async_copy(
                rows[b], out_hbm.at[c, pl.ds(s * PT + (k - 2) * K, K)],
                sems[b]).wait()
        pltpu.sync_copy(acc_s.at[pl.ds(s * PT + k * K, K)], rows[b])
        pltpu.async_copy(rows[b], out_hbm.at[c, pl.ds(s * PT + k * K, K)],
                         sems[b])
    for k in (PT // K - 2, PT // K - 1):
        pltpu.make_async_copy(
            rows[k % 2], out_hbm.at[c, pl.ds(s * PT + k * K, K)],
            sems[k % 2]).wait()


def _feat_body(x_ref, w_ref, b_ref, o_ref):
    acc = lax.dot_general(x_ref[...], w_ref[...], (((1,), (1,)), ((), ())),
                          preferred_element_type=jnp.float32)
    o_ref[...] = jnp.maximum(acc + b_ref[...], 0.0)


_feat_call = pl.pallas_call(
    _feat_body,
    grid=(GRID,),
    in_specs=[
        pl.BlockSpec((RB, D), lambda i: (i, 0)),
        pl.BlockSpec((D, D), lambda i: (0, 0)),
        pl.BlockSpec((D,), lambda i: (0,)),
    ],
    out_specs=pl.BlockSpec((RB, D), lambda i: (i, 0)),
    out_shape=jax.ShapeDtypeStruct((N, D), jnp.float32),
)


def _dinv_slice(dp_ref):
    i = pl.program_id(0)
    d0 = dp_ref[pl.ds(i * RB, RB)]
    d1 = dp_ref[pl.ds(NP + i * RB, RB)]
    return lax.rsqrt(1.0 + d0 + d1)


def _y1_body(h_ref, w_ref, dp_ref, o_ref):
    dv = _dinv_slice(dp_ref)
    acc = lax.dot_general(h_ref[...], w_ref[...], (((1,), (1,)), ((), ())),
                          preferred_element_type=jnp.float32)
    o_ref[...] = acc * dv[:, None]


_y1_call = pl.pallas_call(
    _y1_body,
    grid=(GRID,),
    in_specs=[
        pl.BlockSpec((RB, D), lambda i: (i, 0)),
        pl.BlockSpec((D, D), lambda i: (0, 0)),
        pl.BlockSpec((NC * NP,), lambda i: (0,)),
    ],
    out_specs=pl.BlockSpec((RB, D), lambda i: (i, 0)),
    out_shape=jax.ShapeDtypeStruct((N, D), jnp.float32),
)


def _mid_body(a_ref, y_ref, dp_ref, b_ref, w_ref, o_ref):
    dv = _dinv_slice(dp_ref)
    h = dv[:, None] * (a_ref[0] + a_ref[1] + y_ref[...]) + b_ref[...]
    acc = lax.dot_general(h, w_ref[...], (((1,), (1,)), ((), ())),
                          preferred_element_type=jnp.float32)
    o_ref[...] = acc * dv[:, None]


_mid_call = pl.pallas_call(
    _mid_body,
    grid=(GRID,),
    in_specs=[
        pl.BlockSpec((NC, RB, D), lambda i: (0, i, 0)),
        pl.BlockSpec((RB, D), lambda i: (i, 0)),
        pl.BlockSpec((NC * NP,), lambda i: (0,)),
        pl.BlockSpec((D,), lambda i: (0,)),
        pl.BlockSpec((D, D), lambda i: (0, 0)),
    ],
    out_specs=pl.BlockSpec((RB, D), lambda i: (i, 0)),
    out_shape=jax.ShapeDtypeStruct((N, D), jnp.float32),
)


def _out_body(a_ref, y_ref, dp_ref, b_ref, o_ref):
    dv = _dinv_slice(dp_ref)
    o_ref[...] = dv[:, None] * (a_ref[0] + a_ref[1] + y_ref[...]) + b_ref[...]


_out_call = pl.pallas_call(
    _out_body,
    grid=(GRID,),
    in_specs=[
        pl.BlockSpec((NC, RB, D), lambda i: (0, i, 0)),
        pl.BlockSpec((RB, D), lambda i: (i, 0)),
        pl.BlockSpec((NC * NP,), lambda i: (0,)),
        pl.BlockSpec((D,), lambda i: (0,)),
    ],
    out_specs=pl.BlockSpec((RB, D), lambda i: (i, 0)),
    out_shape=jax.ShapeDtypeStruct((N, D), jnp.float32),
)


def kernel(x, edge_index, W_in, b_in, W1, b1, W2, b2):
    pad = EP - E
    src = jnp.concatenate([edge_index[0], jnp.zeros((pad,), jnp.int32)])
    pad_dst = N + (jnp.arange(pad, dtype=jnp.int32) % (NP - N))
    dst = jnp.concatenate([edge_index[1], pad_dst])
    src3 = jnp.concatenate(
        [src.reshape(NW, CH, K), jnp.zeros((NW, 1, K), jnp.int32)], axis=1)
    dst3 = dst.reshape(NW, CH, K)
    zdeg = jnp.zeros((PT,), jnp.float32)
    ones = jnp.ones((K,), jnp.float32)
    zrows = jnp.zeros((K, D), jnp.float32)

    deg_p = _deg_kernel(dst3, zdeg, ones)
    feat = _feat_call(x, W_in, b_in)

    y1 = _y1_call(feat, W1, deg_p)
    a1 = _edge_kernel(y1, src3, dst3, zrows)
    y2 = _mid_call(a1, y1, deg_p, b1, W2)
    a2 = _edge_kernel(y2, src3, dst3, zrows)
    out = _out_call(a2, y2, deg_p, b2)
    return (feat, out)
